# baseline (speedup 1.0000x reference)
import functools

import jax
import jax.numpy as jnp
from jax import lax
from jax.experimental import pallas as pl
from jax.experimental.pallas import tpu as pltpu
from jax.experimental.pallas import tpu_sc as plsc

_B = 16384
_D = 64
_NC, _NS = 2, 16
_NW = _NC * _NS
_BPW = _B // _NW
_CH = 128
_NCH = _BPW // _CH

_BB = 1024
_FEAT = 128
_H = 128


def _gather_body(uid_hbm, mid_hbm, utab_hbm, mtab_hbm,
                 urows_hbm, mrows_hbm,
                 uidx_v, midx_v, urows_v, mrows_v, sem):
    wid = lax.axis_index("s") * _NC + lax.axis_index("c")
    base = wid * _BPW
    pltpu.sync_copy(uid_hbm.at[pl.ds(wid * _NCH, _NCH)], uidx_v)
    pltpu.sync_copy(mid_hbm.at[pl.ds(wid * _NCH, _NCH)], midx_v)
    copies = []
    for j in range(_NCH):
        copies.append(pltpu.async_copy(
            utab_hbm.at[uidx_v.at[j]], urows_v.at[pl.ds(j * _CH, _CH)], sem))
        copies.append(pltpu.async_copy(
            mtab_hbm.at[midx_v.at[j]], mrows_v.at[pl.ds(j * _CH, _CH)], sem))
    for c in copies:
        c.wait()
    pltpu.sync_copy(urows_v, urows_hbm.at[pl.ds(base, _BPW)])
    pltpu.sync_copy(mrows_v, mrows_hbm.at[pl.ds(base, _BPW)])


@jax.jit
def _sc_gather(user_ids2d, movie_ids2d, user_table, movie_table):
    mesh = plsc.VectorSubcoreMesh(core_axis_name="c", subcore_axis_name="s")
    return pl.kernel(
        _gather_body,
        mesh=mesh,
        out_type=[
            jax.ShapeDtypeStruct((_B, _D), jnp.float32),
            jax.ShapeDtypeStruct((_B, _D), jnp.float32),
        ],
        scratch_types=[
            pltpu.VMEM((_NCH, _CH), jnp.int32),
            pltpu.VMEM((_NCH, _CH), jnp.int32),
            pltpu.VMEM((_BPW, _D), jnp.float32),
            pltpu.VMEM((_BPW, _D), jnp.float32),
            pltpu.SemaphoreType.DMA,
        ],
        compiler_params=pltpu.CompilerParams(use_tc_tiling_on_sc=False),
    )(user_ids2d, movie_ids2d, user_table, movie_table)


def _mlp_body(uv_ref, mv_ref, mf_ref, wf_ref, w1u_ref, w1m_ref, w1f_ref,
              b1_ref, w2_ref, b2_ref, out_ref):
    fv = jnp.dot(mf_ref[...], wf_ref[...], preferred_element_type=jnp.float32)
    acc = jnp.dot(uv_ref[...], w1u_ref[...], preferred_element_type=jnp.float32)
    acc = acc + jnp.dot(mv_ref[...], w1m_ref[...], preferred_element_type=jnp.float32)
    acc = acc + jnp.dot(fv, w1f_ref[...], preferred_element_type=jnp.float32)
    acc = acc + b1_ref[...]
    h = jnp.maximum(acc, 0.0)
    out_ref[...] = jnp.sum(h * w2_ref[...], axis=1) + b2_ref[0, 0]


def _mlp(uv, mv, mf, wf, w1u, w1m, w1f, b1p, w2row, b2):
    grid = (_B // _BB,)
    full = lambda i: (0, 0)
    return pl.pallas_call(
        _mlp_body,
        grid=grid,
        in_specs=[
            pl.BlockSpec((_BB, _D), lambda i: (i, 0)),
            pl.BlockSpec((_BB, _D), lambda i: (i, 0)),
            pl.BlockSpec((_BB, _FEAT), lambda i: (i, 0)),
            pl.BlockSpec((_FEAT, _D), full),
            pl.BlockSpec((_D, _H), full),
            pl.BlockSpec((_D, _H), full),
            pl.BlockSpec((_D, _H), full),
            pl.BlockSpec((1, _H), full),
            pl.BlockSpec((1, _H), full),
            pl.BlockSpec((1, 1), full),
        ],
        out_specs=pl.BlockSpec((_BB,), lambda i: (i,)),
        out_shape=jax.ShapeDtypeStruct((_B,), jnp.float32),
    )(uv, mv, mf, wf, w1u, w1m, w1f, b1p, w2row, b2)


def kernel(user_ids, movie_ids, movie_features, user_table, movie_table,
           W_feat, b_feat, W1, b1, W2, b2):
    uv, mv = _sc_gather(
        user_ids.reshape(_B // _CH, _CH).astype(jnp.int32),
        movie_ids.reshape(_B // _CH, _CH).astype(jnp.int32),
        user_table, movie_table)
    w1u = W1[:_D]
    w1m = W1[_D:2 * _D]
    w1f = W1[2 * _D:]
    b1p = (b1 + b_feat @ w1f).reshape(1, _H)
    out = _mlp(uv, mv, movie_features, W_feat, w1u, w1m, w1f,
               b1p, W2.reshape(1, _H), b2.reshape(1, 1))
    return out


# speedup vs baseline: 1.5685x; 1.5685x over previous
import functools

import jax
import jax.numpy as jnp
from jax import lax
from jax.experimental import pallas as pl
from jax.experimental.pallas import tpu as pltpu
from jax.experimental.pallas import tpu_sc as plsc

_B = 16384
_D = 64
_NC, _NS = 2, 16
_NW = _NC * _NS
_BPW = _B // _NW
_CH = 128
_NCH = _BPW // _CH

_BB = 1024
_FEAT = 128
_H = 128


_K = 16


def _gather_body(uid_hbm, mid_hbm, utab_hbm, mtab_hbm,
                 urows_hbm, mrows_hbm,
                 uidx_v, midx_v, urows_v, mrows_v, sem):
    wid = lax.axis_index("s") * _NC + lax.axis_index("c")
    base = wid * _BPW
    pltpu.sync_copy(uid_hbm.at[pl.ds(base, _BPW)], uidx_v)
    pltpu.sync_copy(mid_hbm.at[pl.ds(base, _BPW)], midx_v)

    half = _BPW // 2

    for p in range(2):
        def chunk(c, _, p=p):
            iv = uidx_v[pl.ds(p * half + c * _K, 16)]
            jv = midx_v[pl.ds(p * half + c * _K, 16)]
            copies = []
            for k in range(_K):
                i = c * _K + k
                copies.append(pltpu.async_copy(
                    utab_hbm.at[pl.ds(iv[k], 1)],
                    urows_v.at[pl.ds(i, 1)], sem))
                copies.append(pltpu.async_copy(
                    mtab_hbm.at[pl.ds(jv[k], 1)],
                    mrows_v.at[pl.ds(i, 1)], sem))
            for cp in copies:
                cp.wait()
            return _

        lax.fori_loop(0, half // _K, chunk, None)
        pltpu.sync_copy(urows_v, urows_hbm.at[pl.ds(base + p * half, half)])
        pltpu.sync_copy(mrows_v, mrows_hbm.at[pl.ds(base + p * half, half)])


@jax.jit
def _sc_gather(user_ids, movie_ids, user_table, movie_table):
    mesh = plsc.VectorSubcoreMesh(core_axis_name="c", subcore_axis_name="s")
    return pl.kernel(
        _gather_body,
        mesh=mesh,
        out_type=[
            jax.ShapeDtypeStruct((_B, _D), jnp.float32),
            jax.ShapeDtypeStruct((_B, _D), jnp.float32),
        ],
        scratch_types=[
            pltpu.VMEM((_BPW,), jnp.int32),
            pltpu.VMEM((_BPW,), jnp.int32),
            pltpu.VMEM((_BPW // 2, _D), jnp.float32),
            pltpu.VMEM((_BPW // 2, _D), jnp.float32),
            pltpu.SemaphoreType.DMA,
        ],
    )(user_ids, movie_ids, user_table, movie_table)


def _mlp_body(uv_ref, mv_ref, mf_ref, wf_ref, w1u_ref, w1m_ref, w1f_ref,
              b1_ref, w2_ref, b2_ref, out_ref):
    fv = jnp.dot(mf_ref[...], wf_ref[...], preferred_element_type=jnp.float32)
    acc = jnp.dot(uv_ref[...], w1u_ref[...], preferred_element_type=jnp.float32)
    acc = acc + jnp.dot(mv_ref[...], w1m_ref[...], preferred_element_type=jnp.float32)
    acc = acc + jnp.dot(fv, w1f_ref[...], preferred_element_type=jnp.float32)
    acc = acc + b1_ref[...]
    h = jnp.maximum(acc, 0.0)
    out_ref[...] = jnp.sum(h * w2_ref[...], axis=1) + b2_ref[0, 0]


def _mlp(uv, mv, mf, wf, w1u, w1m, w1f, b1p, w2row, b2):
    grid = (_B // _BB,)
    full = lambda i: (0, 0)
    return pl.pallas_call(
        _mlp_body,
        grid=grid,
        in_specs=[
            pl.BlockSpec((_BB, _D), lambda i: (i, 0)),
            pl.BlockSpec((_BB, _D), lambda i: (i, 0)),
            pl.BlockSpec((_BB, _FEAT), lambda i: (i, 0)),
            pl.BlockSpec((_FEAT, _D), full),
            pl.BlockSpec((_D, _H), full),
            pl.BlockSpec((_D, _H), full),
            pl.BlockSpec((_D, _H), full),
            pl.BlockSpec((1, _H), full),
            pl.BlockSpec((1, _H), full),
            pl.BlockSpec((1, 1), full),
        ],
        out_specs=pl.BlockSpec((_BB,), lambda i: (i,)),
        out_shape=jax.ShapeDtypeStruct((_B,), jnp.float32),
    )(uv, mv, mf, wf, w1u, w1m, w1f, b1p, w2row, b2)


def kernel(user_ids, movie_ids, movie_features, user_table, movie_table,
           W_feat, b_feat, W1, b1, W2, b2):
    uv, mv = _sc_gather(
        user_ids.astype(jnp.int32), movie_ids.astype(jnp.int32),
        user_table, movie_table)
    w1u = W1[:_D]
    w1m = W1[_D:2 * _D]
    w1f = W1[2 * _D:]
    b1p = (b1 + b_feat @ w1f).reshape(1, _H)
    out = _mlp(uv, mv, movie_features, W_feat, w1u, w1m, w1f,
               b1p, W2.reshape(1, _H), b2.reshape(1, 1))
    return out
